# Initial kernel scaffold; baseline (speedup 1.0000x reference)
import jax
import jax.numpy as jnp
from jax.experimental import pallas as pl


def kernel(attn_log_logits, edge_index):
    raise NotImplementedError("write your pallas kernel here")


# speedup vs baseline: 510.7805x; 510.7805x over previous
import functools

import jax
import jax.numpy as jnp
from jax import lax
from jax.experimental import pallas as pl
from jax.experimental.pallas import tpu as pltpu
from jax.experimental.pallas import tpu_sc as plsc

_N_NODES = 100000
_N_EDGES = 6400000
_PAD_NODES = 102400
_LANES = 16

_NW = 32
_EPW = _N_EDGES // _NW
_C = 4000
_CHUNKS = _EPW // _C


def _sigmoid_table(attn_log_logits):
    x = jnp.pad(attn_log_logits.reshape(-1), (0, _PAD_NODES - _N_NODES))
    x = x.reshape(800, 128)

    def body(x_ref, o_ref):
        o_ref[...] = jax.nn.sigmoid(x_ref[...])

    out = pl.pallas_call(
        body,
        out_shape=jax.ShapeDtypeStruct((800, 128), jnp.float32),
    )(x)
    return out.reshape(-1)


def _edge_attn_sc(table, edges_flat):
    mesh = plsc.VectorSubcoreMesh(core_axis_name="c", subcore_axis_name="s")

    @functools.partial(
        pl.kernel,
        mesh=mesh,
        compiler_params=pltpu.CompilerParams(needs_layout_passes=False),
        out_type=jax.ShapeDtypeStruct((_N_EDGES,), jnp.float32),
        scratch_types=[
            pltpu.VMEM((_PAD_NODES,), jnp.float32),
            pltpu.VMEM((_C,), jnp.int32),
            pltpu.VMEM((_C,), jnp.int32),
            pltpu.VMEM((_C,), jnp.float32),
        ],
    )
    def k(table_hbm, edges_hbm, out_hbm, table_v, sidx_v, didx_v, out_v):
        wid = lax.axis_index("s") * 2 + lax.axis_index("c")
        base_w = wid * _EPW
        pltpu.sync_copy(table_hbm, table_v)

        def chunk_body(kk, carry):
            base = base_w + kk * _C
            pltpu.sync_copy(edges_hbm.at[pl.ds(base, _C)], sidx_v)
            pltpu.sync_copy(edges_hbm.at[pl.ds(_N_EDGES + base, _C)], didx_v)

            def vec_body(i, c2):
                off = i * _LANES
                s = sidx_v[pl.ds(off, _LANES)]
                d = didx_v[pl.ds(off, _LANES)]
                sv = plsc.load_gather(table_v, [s])
                dv = plsc.load_gather(table_v, [d])
                out_v[pl.ds(off, _LANES)] = sv * dv
                return c2

            lax.fori_loop(0, _C // _LANES, vec_body, 0)
            pltpu.sync_copy(out_v, out_hbm.at[pl.ds(base, _C)])
            return carry

        lax.fori_loop(0, _CHUNKS, chunk_body, 0)

    return k(table, edges_flat)


def kernel(attn_log_logits, edge_index):
    table = _sigmoid_table(attn_log_logits)
    edges_flat = edge_index.reshape(-1)
    out = _edge_attn_sc(table, edges_flat)
    return out.reshape(_N_EDGES, 1)


# speedup vs baseline: 631.4813x; 1.2363x over previous
import functools

import jax
import jax.numpy as jnp
from jax import lax
from jax.experimental import pallas as pl
from jax.experimental.pallas import tpu as pltpu
from jax.experimental.pallas import tpu_sc as plsc

_N_NODES = 100000
_N_EDGES = 6400000
_PAD_NODES = 102400
_LANES = 16

_NW = 32
_EPW = _N_EDGES // _NW
_C = 4000
_CHUNKS = _EPW // _C


def _sigmoid_table(attn_log_logits):
    x = jnp.pad(attn_log_logits.reshape(-1), (0, _PAD_NODES - _N_NODES))
    x = x.reshape(800, 128)

    def body(x_ref, o_ref):
        o_ref[...] = jax.nn.sigmoid(x_ref[...])

    out = pl.pallas_call(
        body,
        out_shape=jax.ShapeDtypeStruct((800, 128), jnp.float32),
    )(x)
    return out.reshape(-1)


def _edge_attn_sc(table, edges_flat):
    mesh = plsc.VectorSubcoreMesh(core_axis_name="c", subcore_axis_name="s")

    @functools.partial(
        pl.kernel,
        mesh=mesh,
        compiler_params=pltpu.CompilerParams(needs_layout_passes=False),
        out_type=jax.ShapeDtypeStruct((_N_EDGES,), jnp.float32),
        scratch_types=[
            pltpu.VMEM((_PAD_NODES,), jnp.float32),
            pltpu.VMEM((_C,), jnp.int32),
            pltpu.VMEM((_C,), jnp.int32),
            pltpu.VMEM((_C,), jnp.float32),
        ],
    )
    def k(table_hbm, edges_hbm, out_hbm, table_v, sidx_v, didx_v, out_v):
        wid = lax.axis_index("s") * 2 + lax.axis_index("c")
        base_w = wid * _EPW
        pltpu.sync_copy(table_hbm, table_v)

        def chunk_body(kk, carry):
            base = base_w + kk * _C
            pltpu.sync_copy(edges_hbm.at[pl.ds(base, _C)], sidx_v)
            pltpu.sync_copy(edges_hbm.at[pl.ds(_N_EDGES + base, _C)], didx_v)

            @plsc.parallel_loop(0, _C, step=_LANES, unroll=8)
            def _(off):
                s = sidx_v[pl.ds(off, _LANES)]
                d = didx_v[pl.ds(off, _LANES)]
                sv = plsc.load_gather(table_v, [s])
                dv = plsc.load_gather(table_v, [d])
                out_v[pl.ds(off, _LANES)] = sv * dv
            pltpu.sync_copy(out_v, out_hbm.at[pl.ds(base, _C)])
            return carry

        lax.fori_loop(0, _CHUNKS, chunk_body, 0)

    return k(table, edges_flat)


def kernel(attn_log_logits, edge_index):
    table = _sigmoid_table(attn_log_logits)
    edges_flat = edge_index.reshape(-1)
    out = _edge_attn_sc(table, edges_flat)
    return out.reshape(_N_EDGES, 1)


# speedup vs baseline: 980.4344x; 1.5526x over previous
import functools

import jax
import jax.numpy as jnp
from jax import lax
from jax.experimental import pallas as pl
from jax.experimental.pallas import tpu as pltpu
from jax.experimental.pallas import tpu_sc as plsc

_N_NODES = 100000
_N_EDGES = 6400000
_PAD_NODES = 102400
_LANES = 16

_NW = 32
_EPW = _N_EDGES // _NW
_C = 4000
_CHUNKS = _EPW // _C


def _sigmoid_table(attn_log_logits):
    x = jnp.pad(attn_log_logits.reshape(-1), (0, _PAD_NODES - _N_NODES))
    x = x.reshape(800, 128)

    def body(x_ref, o_ref):
        o_ref[...] = jax.nn.sigmoid(x_ref[...])

    out = pl.pallas_call(
        body,
        out_shape=jax.ShapeDtypeStruct((800, 128), jnp.float32),
    )(x)
    return out.reshape(-1)


def _edge_attn_sc(table, edges_flat):
    mesh = plsc.VectorSubcoreMesh(core_axis_name="c", subcore_axis_name="s")

    @functools.partial(
        pl.kernel,
        mesh=mesh,
        compiler_params=pltpu.CompilerParams(needs_layout_passes=False),
        out_type=jax.ShapeDtypeStruct((_N_EDGES,), jnp.float32),
        scratch_types=[
            pltpu.VMEM((_PAD_NODES,), jnp.float32),
            pltpu.VMEM((_C,), jnp.int32),
            pltpu.VMEM((_C,), jnp.int32),
            pltpu.VMEM((_C,), jnp.int32),
            pltpu.VMEM((_C,), jnp.int32),
            pltpu.VMEM((_C,), jnp.float32),
            pltpu.VMEM((_C,), jnp.float32),
            pltpu.SemaphoreType.DMA,
            pltpu.SemaphoreType.DMA,
            pltpu.SemaphoreType.DMA,
            pltpu.SemaphoreType.DMA,
        ],
    )
    def k(table_hbm, edges_hbm, out_hbm, table_v,
          si0, si1, di0, di1, o0, o1, smi0, smi1, smo0, smo1):
        wid = lax.axis_index("s") * 2 + lax.axis_index("c")
        base_w = wid * _EPW
        si, di, o = (si0, si1), (di0, di1), (o0, o1)
        smi, smo = (smi0, smi1), (smo0, smo1)
        pltpu.sync_copy(table_hbm, table_v)

        def start_in(kk, b):
            base = base_w + kk * _C
            pltpu.async_copy(edges_hbm.at[pl.ds(base, _C)], si[b], smi[b])
            pltpu.async_copy(
                edges_hbm.at[pl.ds(_N_EDGES + base, _C)], di[b], smi[b])

        def wait_in(b):
            pltpu.make_async_copy(
                edges_hbm.at[pl.ds(0, _C)], si[b], smi[b]).wait()
            pltpu.make_async_copy(
                edges_hbm.at[pl.ds(0, _C)], di[b], smi[b]).wait()

        def wait_out(b):
            pltpu.make_async_copy(
                o[b], out_hbm.at[pl.ds(0, _C)], smo[b]).wait()

        start_in(0, 0)

        def pair_body(p, carry):
            for b in range(2):
                kk = p * 2 + b
                wait_in(b)

                @pl.when(kk + 1 < _CHUNKS)
                def _():
                    start_in(kk + 1, 1 - b)

                @pl.when(kk >= 2)
                def _():
                    wait_out(b)

                @plsc.parallel_loop(0, _C, step=_LANES, unroll=8)
                def _(off):
                    s = si[b][pl.ds(off, _LANES)]
                    d = di[b][pl.ds(off, _LANES)]
                    sv = plsc.load_gather(table_v, [s])
                    dv = plsc.load_gather(table_v, [d])
                    o[b][pl.ds(off, _LANES)] = sv * dv

                pltpu.async_copy(
                    o[b], out_hbm.at[pl.ds(base_w + kk * _C, _C)], smo[b])
            return carry

        lax.fori_loop(0, _CHUNKS // 2, pair_body, 0)
        wait_out(0)
        wait_out(1)

    return k(table, edges_flat)


def kernel(attn_log_logits, edge_index):
    table = _sigmoid_table(attn_log_logits)
    edges_flat = edge_index.reshape(-1)
    out = _edge_attn_sc(table, edges_flat)
    return out.reshape(_N_EDGES, 1)


# speedup vs baseline: 1275.0544x; 1.3005x over previous
import functools

import jax
import jax.numpy as jnp
from jax import lax
from jax.experimental import pallas as pl
from jax.experimental.pallas import tpu as pltpu
from jax.experimental.pallas import tpu_sc as plsc

_N_NODES = 100000
_N_EDGES = 6400000
_PAD_NODES = 102400
_LANES = 16

_NW = 32
_C = 3200
_CHUNKS = _N_EDGES // _C
_MAX_PAIRS = (_CHUNKS // _NW + 2) // 2


def _sigmoid_table(attn_log_logits):
    x = jnp.pad(attn_log_logits.reshape(-1), (0, _PAD_NODES - _N_NODES))
    x = x.reshape(800, 128)

    def body(x_ref, o_ref):
        o_ref[...] = jax.nn.sigmoid(x_ref[...])

    out = pl.pallas_call(
        body,
        out_shape=jax.ShapeDtypeStruct((800, 128), jnp.float32),
    )(x)
    return out.reshape(-1)


def _edge_attn_sc(table, edge_index):
    mesh = plsc.VectorSubcoreMesh(core_axis_name="c", subcore_axis_name="s")

    @functools.partial(
        pl.kernel,
        mesh=mesh,
        compiler_params=pltpu.CompilerParams(needs_layout_passes=False),
        out_type=jax.ShapeDtypeStruct((_N_EDGES,), jnp.float32),
        scratch_types=[
            pltpu.VMEM((_PAD_NODES,), jnp.float32),
            pltpu.VMEM((2, _C), jnp.int32),
            pltpu.VMEM((2, _C), jnp.int32),
            pltpu.VMEM((_C,), jnp.float32),
            pltpu.VMEM((_C,), jnp.float32),
            pltpu.SemaphoreType.DMA,
            pltpu.SemaphoreType.DMA,
            pltpu.SemaphoreType.DMA,
            pltpu.SemaphoreType.DMA,
        ],
    )
    def k(table_hbm, edges_hbm, out_hbm, table_v,
          e0, e1, o0, o1, smi0, smi1, smo0, smo1):
        wid = lax.axis_index("s") * 2 + lax.axis_index("c")
        e, o = (e0, e1), (o0, o1)
        smi, smo = (smi0, smi1), (smo0, smo1)
        pltpu.sync_copy(table_hbm, table_v)

        def start_in(c, b):
            pltpu.async_copy(
                edges_hbm.at[:, pl.ds(c * _C, _C)], e[b], smi[b])

        def wait_in(b):
            pltpu.make_async_copy(
                edges_hbm.at[:, pl.ds(0, _C)], e[b], smi[b]).wait()

        def wait_out(b):
            pltpu.make_async_copy(
                o[b], out_hbm.at[pl.ds(0, _C)], smo[b]).wait()

        start_in(wid, 0)

        def pair_body(p, carry):
            for b in range(2):
                j = p * 2 + b
                c = wid + j * _NW

                @pl.when(c < _CHUNKS)
                def _():
                    wait_in(b)

                    @pl.when(c + _NW < _CHUNKS)
                    def _():
                        start_in(c + _NW, 1 - b)

                    @pl.when(j >= 2)
                    def _():
                        wait_out(b)

                    @plsc.parallel_loop(0, _C, step=_LANES, unroll=8)
                    def _(off):
                        s = e[b][0, pl.ds(off, _LANES)]
                        d = e[b][1, pl.ds(off, _LANES)]
                        sv = plsc.load_gather(table_v, [s])
                        dv = plsc.load_gather(table_v, [d])
                        o[b][pl.ds(off, _LANES)] = sv * dv

                    pltpu.async_copy(
                        o[b], out_hbm.at[pl.ds(c * _C, _C)], smo[b])
            return carry

        lax.fori_loop(0, _MAX_PAIRS, pair_body, 0)
        wait_out(0)
        wait_out(1)

    return k(table, edge_index)


def kernel(attn_log_logits, edge_index):
    table = _sigmoid_table(attn_log_logits)
    out = _edge_attn_sc(table, edge_index)
    return out.reshape(_N_EDGES, 1)


# speedup vs baseline: 1532.3713x; 1.2018x over previous
import functools

import jax
import jax.numpy as jnp
from jax import lax
from jax.experimental import pallas as pl
from jax.experimental.pallas import tpu as pltpu
from jax.experimental.pallas import tpu_sc as plsc

_N_NODES = 100000
_N_EDGES = 6400000
_PAD_NODES = 102400
_LANES = 16

_NW = 32
_C = 5120
_CHUNKS = _N_EDGES // _C
_MAX_PAIRS = (_CHUNKS // _NW + 2) // 2
_TBL = 100096


def _sigmoid_table(attn_log_logits):
    x = jnp.pad(attn_log_logits.reshape(-1), (0, _PAD_NODES - _N_NODES))
    x = x.reshape(800, 128)

    def body(x_ref, o_ref):
        o_ref[...] = jax.nn.sigmoid(x_ref[...])

    out = pl.pallas_call(
        body,
        out_shape=jax.ShapeDtypeStruct((800, 128), jnp.float32),
    )(x)
    return out.reshape(-1)


def _edge_attn_sc(table, edge_index):
    mesh = plsc.VectorSubcoreMesh(core_axis_name="c", subcore_axis_name="s")

    @functools.partial(
        pl.kernel,
        mesh=mesh,
        compiler_params=pltpu.CompilerParams(needs_layout_passes=False),
        out_type=jax.ShapeDtypeStruct((_N_EDGES,), jnp.float32),
        scratch_types=[
            pltpu.VMEM((_TBL,), jnp.float32),
            pltpu.VMEM((2, _C), jnp.int32),
            pltpu.VMEM((2, _C), jnp.int32),
            pltpu.VMEM((_C,), jnp.float32),
            pltpu.VMEM((_C,), jnp.float32),
            pltpu.SemaphoreType.DMA,
            pltpu.SemaphoreType.DMA,
            pltpu.SemaphoreType.DMA,
            pltpu.SemaphoreType.DMA,
        ],
    )
    def k(table_hbm, edges_hbm, out_hbm, table_v,
          e0, e1, o0, o1, smi0, smi1, smo0, smo1):
        wid = lax.axis_index("s") * 2 + lax.axis_index("c")
        e, o = (e0, e1), (o0, o1)
        smi, smo = (smi0, smi1), (smo0, smo1)

        def start_in(c, b):
            pltpu.async_copy(
                edges_hbm.at[:, pl.ds(c * _C, _C)], e[b], smi[b])

        def wait_in(b):
            pltpu.make_async_copy(
                edges_hbm.at[:, pl.ds(0, _C)], e[b], smi[b]).wait()

        def wait_out(b):
            pltpu.make_async_copy(
                o[b], out_hbm.at[pl.ds(0, _C)], smo[b]).wait()

        start_in(wid, 0)
        pltpu.sync_copy(table_hbm.at[pl.ds(0, _TBL)], table_v)

        def pair_body(p, carry):
            for b in range(2):
                j = p * 2 + b
                c = wid + j * _NW

                @pl.when(c < _CHUNKS)
                def _():
                    wait_in(b)

                    @pl.when(c + _NW < _CHUNKS)
                    def _():
                        start_in(c + _NW, 1 - b)

                    @pl.when(j >= 2)
                    def _():
                        wait_out(b)

                    @plsc.parallel_loop(0, _C, step=_LANES, unroll=16)
                    def _(off):
                        s = e[b][0, pl.ds(off, _LANES)]
                        d = e[b][1, pl.ds(off, _LANES)]
                        sv = plsc.load_gather(table_v, [s])
                        dv = plsc.load_gather(table_v, [d])
                        o[b][pl.ds(off, _LANES)] = sv * dv

                    pltpu.async_copy(
                        o[b], out_hbm.at[pl.ds(c * _C, _C)], smo[b])
            return carry

        lax.fori_loop(0, _MAX_PAIRS, pair_body, 0)
        wait_out(0)
        wait_out(1)

    return k(table, edge_index)


def kernel(attn_log_logits, edge_index):
    table = _sigmoid_table(attn_log_logits)
    out = _edge_attn_sc(table, edge_index)
    return out.reshape(_N_EDGES, 1)


# speedup vs baseline: 1636.5161x; 1.0680x over previous
import functools

import jax
import jax.numpy as jnp
from jax import lax
from jax.experimental import pallas as pl
from jax.experimental.pallas import tpu as pltpu
from jax.experimental.pallas import tpu_sc as plsc

_N_NODES = 100000
_N_EDGES = 6400000
_PAD_NODES = 102400
_LANES = 16

_NW = 32
_C = 5120
_CHUNKS = _N_EDGES // _C
_MAX_PAIRS = (_CHUNKS // _NW + 2) // 2
_TBL = 100096


def _sigmoid_table(attn_log_logits):
    x = jnp.pad(attn_log_logits.reshape(-1), (0, _PAD_NODES - _N_NODES))
    x = x.reshape(800, 128)

    def body(x_ref, o_ref):
        o_ref[...] = jax.nn.sigmoid(x_ref[...])

    out = pl.pallas_call(
        body,
        out_shape=jax.ShapeDtypeStruct((800, 128), jnp.float32),
    )(x)
    return out.reshape(-1)


def _edge_attn_sc(table, edge_index):
    mesh = plsc.VectorSubcoreMesh(core_axis_name="c", subcore_axis_name="s")

    @functools.partial(
        pl.kernel,
        mesh=mesh,
        compiler_params=pltpu.CompilerParams(needs_layout_passes=False),
        out_type=jax.ShapeDtypeStruct((_N_EDGES,), jnp.float32),
        scratch_types=[
            pltpu.VMEM((_TBL,), jnp.float32),
            pltpu.VMEM((2, _C), jnp.int32),
            pltpu.VMEM((2, _C), jnp.int32),
            pltpu.VMEM((_C,), jnp.float32),
            pltpu.VMEM((_C,), jnp.float32),
            pltpu.SemaphoreType.DMA,
            pltpu.SemaphoreType.DMA,
            pltpu.SemaphoreType.DMA,
            pltpu.SemaphoreType.DMA,
        ],
    )
    def k(table_hbm, edges_hbm, out_hbm, table_v,
          e0, e1, o0, o1, smi0, smi1, smo0, smo1):
        wid = lax.axis_index("s") * 2 + lax.axis_index("c")
        e, o = (e0, e1), (o0, o1)
        smi, smo = (smi0, smi1), (smo0, smo1)

        def start_in(c, b):
            pltpu.async_copy(
                edges_hbm.at[:, pl.ds(c * _C, _C)], e[b], smi[b])

        def wait_in(b):
            pltpu.make_async_copy(
                edges_hbm.at[:, pl.ds(0, _C)], e[b], smi[b]).wait()

        def wait_out(b):
            pltpu.make_async_copy(
                o[b], out_hbm.at[pl.ds(0, _C)], smo[b]).wait()

        start_in(wid, 0)
        start_in(wid + _NW, 1)
        pltpu.sync_copy(table_hbm.at[pl.ds(0, _TBL)], table_v)

        def pair_body(p, carry):
            for b in range(2):
                j = p * 2 + b
                c = wid + j * _NW

                @pl.when(c < _CHUNKS)
                def _():
                    wait_in(b)

                    @pl.when(j >= 2)
                    def _():
                        wait_out(b)

                    @plsc.parallel_loop(0, _C, step=_LANES, unroll=16)
                    def _(off):
                        s = e[b][0, pl.ds(off, _LANES)]
                        d = e[b][1, pl.ds(off, _LANES)]
                        sv = plsc.load_gather(table_v, [s])
                        dv = plsc.load_gather(table_v, [d])
                        o[b][pl.ds(off, _LANES)] = sv * dv

                    pltpu.async_copy(
                        o[b], out_hbm.at[pl.ds(c * _C, _C)], smo[b])

                    @pl.when(c + 2 * _NW < _CHUNKS)
                    def _():
                        start_in(c + 2 * _NW, b)
            return carry

        lax.fori_loop(0, _MAX_PAIRS, pair_body, 0)
        wait_out(0)
        wait_out(1)

    return k(table, edge_index)


def kernel(attn_log_logits, edge_index):
    table = _sigmoid_table(attn_log_logits)
    out = _edge_attn_sc(table, edge_index)
    return out.reshape(_N_EDGES, 1)
